# baseline (speedup 1.0000x reference)
import jax
import jax.numpy as jnp
from jax.experimental import pallas as pl
from jax.experimental.pallas import tpu as pltpu

MAX_LORAS = 8
MAX_RANK = 64
TM = 512


def _body(tok_ref, eff_ref, scal_ref, x_ref, a_ref, b_ref, o_ref,
          a_bf, b_bf):
    i = pl.program_id(0)
    E, r = MAX_LORAS, MAX_RANK
    R = E * r

    @pl.when(i == 0)
    def _prep():
        row = jax.lax.broadcasted_iota(jnp.int32, (R, 1), 0)
        slot_of_row = jax.lax.shift_right_logical(row, 6)
        r_of_row = jnp.bitwise_and(row, r - 1)
        eff_row = jnp.zeros((R, 1), jnp.int32)
        scal_row = jnp.zeros((R, 1), jnp.float32)
        for s in range(E):
            sel = slot_of_row == s
            eff_row = jnp.where(sel, eff_ref[s], eff_row)
            scal_row = jnp.where(sel, scal_ref[s], scal_row)
        w = jnp.where(r_of_row < eff_row, scal_row, 0.0)
        a_bf[...] = (a_ref[...] * w).astype(jnp.bfloat16)
        for s in range(E):
            b_bf[:, s * r:(s + 1) * r] = b_ref[s].astype(jnp.bfloat16)

    xb = x_ref[...].astype(jnp.bfloat16)
    h = jax.lax.dot_general(
        xb, a_bf[...], (((1,), (1,)), ((), ())),
        preferred_element_type=jnp.float32)
    col = jax.lax.broadcasted_iota(jnp.int32, (1, R), 1)
    slot_of_col = jax.lax.shift_right_logical(col, 6)
    tok = tok_ref[...]
    mask = slot_of_col == tok[:, None]
    h = jnp.where(mask, h.astype(jnp.bfloat16), jnp.bfloat16(0.0))
    o_ref[...] = jax.lax.dot_general(
        h, b_bf[...], (((1,), (1,)), ((), ())),
        preferred_element_type=jnp.float32)


def kernel(x, lora_a, lora_b, lora_scaling, effective_rank, token_to_slot):
    T, d_in = x.shape
    E, r, _ = lora_a.shape
    d_out = lora_b.shape[1]
    R = E * r
    nblk = T // TM

    tok = token_to_slot.astype(jnp.int32)
    a_cat = lora_a.reshape(R, d_in)

    out = pl.pallas_call(
        _body,
        grid=(nblk,),
        in_specs=[
            pl.BlockSpec((TM,), lambda i: (i,)),
            pl.BlockSpec((E,), lambda i: (0,)),
            pl.BlockSpec((E,), lambda i: (0,)),
            pl.BlockSpec((TM, d_in), lambda i: (i, 0)),
            pl.BlockSpec((R, d_in), lambda i: (0, 0)),
            pl.BlockSpec((E, d_out, r), lambda i: (0, 0, 0)),
        ],
        out_specs=pl.BlockSpec((TM, d_out), lambda i: (i, 0)),
        out_shape=jax.ShapeDtypeStruct((T, d_out), jnp.float32),
        scratch_shapes=[
            pltpu.VMEM((R, d_in), jnp.bfloat16),
            pltpu.VMEM((d_out, R), jnp.bfloat16),
        ],
        compiler_params=pltpu.CompilerParams(
            dimension_semantics=("arbitrary",)),
    )(tok, effective_rank, lora_scaling, x, a_cat, lora_b)
    return out


# speedup vs baseline: 1.0199x; 1.0199x over previous
import jax
import jax.numpy as jnp
from jax.experimental import pallas as pl
from jax.experimental.pallas import tpu as pltpu

MAX_LORAS = 8
MAX_RANK = 64
TM = 1024


def _body(tok_ref, eff_ref, scal_ref, x_ref, a_ref, b_ref, o_ref,
          a_bf, b_bf):
    i = pl.program_id(0)
    E, r = MAX_LORAS, MAX_RANK
    R = E * r

    @pl.when(i == 0)
    def _prep():
        row = jax.lax.broadcasted_iota(jnp.int32, (R, 1), 0)
        slot_of_row = jax.lax.shift_right_logical(row, 6)
        r_of_row = jnp.bitwise_and(row, r - 1)
        eff_row = jnp.zeros((R, 1), jnp.int32)
        scal_row = jnp.zeros((R, 1), jnp.float32)
        for s in range(E):
            sel = slot_of_row == s
            eff_row = jnp.where(sel, eff_ref[s], eff_row)
            scal_row = jnp.where(sel, scal_ref[s], scal_row)
        w = jnp.where(r_of_row < eff_row, scal_row, 0.0)
        a_bf[...] = (a_ref[...] * w).astype(jnp.bfloat16)
        for s in range(E):
            b_bf[:, s * r:(s + 1) * r] = b_ref[s].astype(jnp.bfloat16)

    xb = x_ref[...].astype(jnp.bfloat16)
    h = jax.lax.dot_general(
        xb, a_bf[...], (((1,), (1,)), ((), ())),
        preferred_element_type=jnp.float32)
    col = jax.lax.broadcasted_iota(jnp.int32, (1, R), 1)
    slot_of_col = jax.lax.shift_right_logical(col, 6)
    tok = tok_ref[...]
    mask = slot_of_col == tok[:, None]
    h = jnp.where(mask, h.astype(jnp.bfloat16), jnp.bfloat16(0.0))
    o_ref[...] = jax.lax.dot_general(
        h, b_bf[...], (((1,), (1,)), ((), ())),
        preferred_element_type=jnp.float32)


def kernel(x, lora_a, lora_b, lora_scaling, effective_rank, token_to_slot):
    T, d_in = x.shape
    E, r, _ = lora_a.shape
    d_out = lora_b.shape[1]
    R = E * r
    nblk = T // TM

    tok = token_to_slot.astype(jnp.int32)
    a_cat = lora_a.reshape(R, d_in)

    out = pl.pallas_call(
        _body,
        grid=(nblk,),
        in_specs=[
            pl.BlockSpec((TM,), lambda i: (i,)),
            pl.BlockSpec((E,), lambda i: (0,)),
            pl.BlockSpec((E,), lambda i: (0,)),
            pl.BlockSpec((TM, d_in), lambda i: (i, 0)),
            pl.BlockSpec((R, d_in), lambda i: (0, 0)),
            pl.BlockSpec((E, d_out, r), lambda i: (0, 0, 0)),
        ],
        out_specs=pl.BlockSpec((TM, d_out), lambda i: (i, 0)),
        out_shape=jax.ShapeDtypeStruct((T, d_out), jnp.float32),
        scratch_shapes=[
            pltpu.VMEM((R, d_in), jnp.bfloat16),
            pltpu.VMEM((d_out, R), jnp.bfloat16),
        ],
        compiler_params=pltpu.CompilerParams(
            dimension_semantics=("arbitrary",)),
    )(tok, effective_rank, lora_scaling, x, a_cat, lora_b)
    return out


# speedup vs baseline: 1.0465x; 1.0261x over previous
import jax
import jax.numpy as jnp
from jax.experimental import pallas as pl

MAX_LORAS = 8
MAX_RANK = 64
TM = 1024


def _body(tok_ref, effc_ref, scalc_ref, x_ref, a_ref, b_ref, o_ref):
    xb = x_ref[...].astype(jnp.bfloat16)
    h = jax.lax.dot_general(
        xb, a_ref[...].astype(jnp.bfloat16), (((1,), (1,)), ((), ())),
        preferred_element_type=jnp.float32)
    R = MAX_LORAS * MAX_RANK
    col = jax.lax.broadcasted_iota(jnp.int32, (TM, R), 1)
    slot_of_col = jax.lax.shift_right_logical(col, 6)
    r_of_col = jnp.bitwise_and(col, MAX_RANK - 1)
    tok = tok_ref[0, 0, :]
    mask = (slot_of_col == tok[:, None]) & (r_of_col < effc_ref[0, :][None, :])
    h = jnp.where(mask, h * scalc_ref[0, :][None, :], 0.0)
    o_ref[...] = jax.lax.dot_general(
        h.astype(jnp.bfloat16), b_ref[...].astype(jnp.bfloat16),
        (((1,), (1,)), ((), ())),
        preferred_element_type=jnp.float32)


def kernel(x, lora_a, lora_b, lora_scaling, effective_rank, token_to_slot):
    T, d_in = x.shape
    E, r, _ = lora_a.shape
    d_out = lora_b.shape[1]
    R = E * r
    nblk = T // TM

    tok = token_to_slot.astype(jnp.int32).reshape(nblk, 1, TM)
    a_cat = lora_a.reshape(R, d_in)
    b_cat = lora_b.transpose(1, 0, 2).reshape(d_out, R)
    eff_cols = jnp.repeat(effective_rank, r).reshape(1, R)
    scal_cols = jnp.repeat(lora_scaling, r).reshape(1, R)

    out = pl.pallas_call(
        _body,
        grid=(nblk,),
        in_specs=[
            pl.BlockSpec((1, 1, TM), lambda i: (i, 0, 0)),
            pl.BlockSpec((1, R), lambda i: (0, 0)),
            pl.BlockSpec((1, R), lambda i: (0, 0)),
            pl.BlockSpec((TM, d_in), lambda i: (i, 0)),
            pl.BlockSpec((R, d_in), lambda i: (0, 0)),
            pl.BlockSpec((d_out, R), lambda i: (0, 0)),
        ],
        out_specs=pl.BlockSpec((TM, d_out), lambda i: (i, 0)),
        out_shape=jax.ShapeDtypeStruct((T, d_out), jnp.float32),
    )(tok, eff_cols, scal_cols, x, a_cat, b_cat)
    return out
